# Initial kernel scaffold; baseline (speedup 1.0000x reference)
import jax
import jax.numpy as jnp
from jax.experimental import pallas as pl


def kernel(x_maccs, x_estate, x_attrmask, linear_w_0, linear_w_1, linear_w_2, linear_b_0, linear_b_1, linear_b_2, conv_w, conv_b):
    raise NotImplementedError("write your pallas kernel here")


# speedup vs baseline: 3.5034x; 3.5034x over previous
import jax
import jax.numpy as jnp
from jax.experimental import pallas as pl
from jax.experimental.pallas import tpu as pltpu


_TILE_B = 1024


def _fused_kernel(x0_ref, x1_ref, x2_ref,
                  w0t_ref, w1t_ref, w2t_ref,
                  wc0_ref, wc1_ref, wc2_ref,
                  b0_ref, b1_ref, b2_ref, cbr_ref,
                  o_ref):
    m0 = jnp.dot(w0t_ref[...], wc0_ref[...], preferred_element_type=jnp.float32)
    m1 = jnp.dot(w1t_ref[...], wc1_ref[...], preferred_element_type=jnp.float32)
    m2 = jnp.dot(w2t_ref[...], wc2_ref[...], preferred_element_type=jnp.float32)
    beta = (cbr_ref[...]
            + jnp.dot(b0_ref[...], wc0_ref[...], preferred_element_type=jnp.float32)
            + jnp.dot(b1_ref[...], wc1_ref[...], preferred_element_type=jnp.float32)
            + jnp.dot(b2_ref[...], wc2_ref[...], preferred_element_type=jnp.float32))
    acc = jnp.dot(x0_ref[...], m0, preferred_element_type=jnp.float32)
    acc = acc + jnp.dot(x1_ref[...], m1, preferred_element_type=jnp.float32)
    acc = acc + jnp.dot(x2_ref[...], m2, preferred_element_type=jnp.float32)
    o_ref[...] = (acc + beta).astype(o_ref.dtype)


def _conv_tap_matrix(conv_w, e):
    O, C, K = conv_w.shape
    cols = []
    for l in range(3):
        k = e - l + 1
        if 0 <= k < K:
            cols.append(jnp.transpose(conv_w[:, :, k]))
        else:
            cols.append(jnp.zeros((C, O), conv_w.dtype))
    return jnp.stack(cols, axis=-1).reshape(C, O * 3)


def kernel(x_maccs, x_estate, x_attrmask,
           linear_w_0, linear_w_1, linear_w_2,
           linear_b_0, linear_b_1, linear_b_2,
           conv_w, conv_b):
    B = x_maccs.shape[0]
    D0 = x_maccs.shape[1]
    D1 = x_estate.shape[1]
    D2 = x_attrmask.shape[1]
    O = conv_w.shape[0]
    N = O * 3

    f32 = jnp.float32
    w0t = jnp.transpose(linear_w_0).astype(f32)
    w1t = jnp.transpose(linear_w_1).astype(f32)
    w2t = jnp.transpose(linear_w_2).astype(f32)
    wc0 = _conv_tap_matrix(conv_w.astype(f32), 0)
    wc1 = _conv_tap_matrix(conv_w.astype(f32), 1)
    wc2 = _conv_tap_matrix(conv_w.astype(f32), 2)
    b0 = linear_b_0.reshape(1, -1).astype(f32)
    b1 = linear_b_1.reshape(1, -1).astype(f32)
    b2 = linear_b_2.reshape(1, -1).astype(f32)
    cbr = jnp.repeat(conv_b.astype(f32), 3).reshape(1, N)

    tm = min(_TILE_B, B)
    grid = pl.cdiv(B, tm)
    C = w0t.shape[1]

    out = pl.pallas_call(
        _fused_kernel,
        out_shape=jax.ShapeDtypeStruct((B, N), f32),
        grid_spec=pltpu.PrefetchScalarGridSpec(
            num_scalar_prefetch=0,
            grid=(grid,),
            in_specs=[
                pl.BlockSpec((tm, D0), lambda i: (i, 0)),
                pl.BlockSpec((tm, D1), lambda i: (i, 0)),
                pl.BlockSpec((tm, D2), lambda i: (i, 0)),
                pl.BlockSpec((D0, C), lambda i: (0, 0)),
                pl.BlockSpec((D1, C), lambda i: (0, 0)),
                pl.BlockSpec((D2, C), lambda i: (0, 0)),
                pl.BlockSpec((C, N), lambda i: (0, 0)),
                pl.BlockSpec((C, N), lambda i: (0, 0)),
                pl.BlockSpec((C, N), lambda i: (0, 0)),
                pl.BlockSpec((1, C), lambda i: (0, 0)),
                pl.BlockSpec((1, C), lambda i: (0, 0)),
                pl.BlockSpec((1, C), lambda i: (0, 0)),
                pl.BlockSpec((1, N), lambda i: (0, 0)),
            ],
            out_specs=pl.BlockSpec((tm, N), lambda i: (i, 0)),
        ),
        compiler_params=pltpu.CompilerParams(
            dimension_semantics=("parallel",)),
    )(x_maccs.astype(f32), x_estate.astype(f32), x_attrmask.astype(f32),
      w0t, w1t, w2t, wc0, wc1, wc2, b0, b1, b2, cbr)
    return out
